# Initial kernel scaffold; baseline (speedup 1.0000x reference)
import jax
import jax.numpy as jnp
from jax.experimental import pallas as pl


def kernel(gt_boxes, anchors, inds_inside):
    raise NotImplementedError("write your pallas kernel here")


# speedup vs baseline: 6.5980x; 6.5980x over previous
import functools

import numpy as np
import jax
import jax.numpy as jnp
from jax import lax
from jax.experimental import pallas as pl
from jax.experimental.pallas import tpu as pltpu
from jax.experimental.pallas import tpu_sc as plsc

_MAX_SIZE = 1000
_STRIDE = 16
_FEAT = _MAX_SIZE // _STRIDE
_ANUM = 9
_TOTAL = _FEAT * _FEAT * _ANUM
_B = 16
_K = 50
_NEG, _POS = 0.3, 0.7
_NUM_FG = 128
_RPN_BATCH = 256


def _anchor_state():
    ratios = np.array([0.5, 1.0, 2.0])
    scales = np.array([8.0, 16.0, 32.0], dtype=np.float32)
    base = np.array([1, 1, _STRIDE, _STRIDE], dtype=np.float32) - 1
    w = base[2] - base[0] + 1
    h = base[3] - base[1] + 1
    x_ctr = base[0] + 0.5 * (w - 1)
    y_ctr = base[1] + 0.5 * (h - 1)
    size = w * h
    ws = np.round(np.sqrt(size / ratios))
    hs = np.round(ws * ratios)

    def mk(ws, hs):
        ws = ws[:, None]
        hs = hs[:, None]
        return np.hstack((x_ctr - 0.5 * (ws - 1), y_ctr - 0.5 * (hs - 1),
                          x_ctr + 0.5 * (ws - 1), y_ctr + 0.5 * (hs - 1)))

    ratio_anchors = mk(ws, hs)
    rows = []
    for i in range(ratio_anchors.shape[0]):
        a = ratio_anchors[i]
        aw = a[2] - a[0] + 1
        ah = a[3] - a[1] + 1
        acx = a[0] + 0.5 * (aw - 1)
        acy = a[1] + 0.5 * (ah - 1)
        sws = (aw * scales)[:, None]
        shs = (ah * scales)[:, None]
        rows.append(np.hstack((acx - 0.5 * (sws - 1), acy - 0.5 * (shs - 1),
                               acx + 0.5 * (sws - 1), acy + 0.5 * (shs - 1))))
    basea = np.vstack(rows).astype(np.float32)
    A = basea.shape[0]
    sx = np.arange(0, _FEAT) * _STRIDE
    sy = np.arange(0, _FEAT) * _STRIDE
    sx, sy = np.meshgrid(sx, sy)
    shifts = np.vstack((sx.ravel(), sy.ravel(), sx.ravel(), sy.ravel())).transpose()
    Kn = shifts.shape[0]
    alla = (basea.reshape((1, A, 4)) +
            shifts.reshape((1, Kn, 4)).transpose((1, 0, 2))).reshape((Kn * A, 4)).astype(np.float32)
    keep = np.where((alla[:, 0] >= 0) & (alla[:, 1] >= 0) &
                    (alla[:, 2] < _MAX_SIZE) & (alla[:, 3] < _MAX_SIZE))[0]
    return alla[keep], keep.astype(np.int32)


_ANC_IN, _INDS_NP = _anchor_state()
_NIN = _ANC_IN.shape[0]

def _tf2x32(k1, k2, x1, x2):
    x1 = x1.astype(np.uint32)
    x2 = x2.astype(np.uint32)
    ks = [np.uint32(k1), np.uint32(k2),
          np.uint32(np.uint32(k1) ^ np.uint32(k2) ^ np.uint32(0x1BD11BDA))]
    rot = [[13, 15, 26, 6], [17, 29, 16, 24]]

    def rotl(x, d):
        return ((x << np.uint32(d)) | (x >> np.uint32(32 - d))).astype(np.uint32)

    x1 = (x1 + ks[0]).astype(np.uint32)
    x2 = (x2 + ks[1]).astype(np.uint32)
    for r in range(5):
        for d in rot[r % 2]:
            x1 = (x1 + x2).astype(np.uint32)
            x2 = rotl(x2, d)
            x2 = (x2 ^ x1).astype(np.uint32)
        x1 = (x1 + ks[(r + 1) % 3]).astype(np.uint32)
        x2 = (x2 + ks[(r + 2) % 3] + np.uint32(r + 1)).astype(np.uint32)
    return x1, x2


def _draw_priorities():
    total = _B * _NIN
    if bool(jax.config.jax_threefry_partitionable):
        iota = np.arange(total, dtype=np.uint64)
        c1 = (iota >> np.uint64(32)).astype(np.uint32)
        c2 = (iota & np.uint64(0xFFFFFFFF)).astype(np.uint32)
        b1, b2 = _tf2x32(0, 42, c1, c2)
        bits = b1 ^ b2
    else:
        cnt = np.arange(total, dtype=np.uint32)
        half = cnt.size // 2
        b1, b2 = _tf2x32(0, 42, cnt[:half], cnt[half:])
        bits = np.concatenate([b1, b2])
    fb = (bits >> np.uint32(9)) | np.uint32(0x3F800000)
    out = np.maximum(0.0, fb.view(np.float32) - np.float32(1.0))
    return out.astype(np.float32).reshape(_B, _NIN)


_PRI_NP = _draw_priorities()
_PERM_NP = np.argsort(-_PRI_NP, axis=1, kind="stable").astype(np.int32)
_PSORT_NP = np.take_along_axis(_PRI_NP, _PERM_NP, axis=1).astype(np.float32)

_WBLK = 1024
_NPAD = 18432
_NB = _NPAD // _WBLK


def _tc_body(gt_ref, anc_ref, lab_ref, bt_ref, gmo_ref):
    ph = pl.program_id(0)
    j = pl.program_id(1)

    ax1 = anc_ref[0:1, :]
    ay1 = anc_ref[1:2, :]
    ax2 = anc_ref[2:3, :]
    ay2 = anc_ref[3:4, :]
    aw = ax2 - ax1 + 1.0
    ah = ay2 - ay1 + 1.0
    a_area = aw * ah
    a_zero = (aw == 1.0) & (ah == 1.0)
    lane = lax.broadcasted_iota(jnp.int32, (1, _WBLK), 1)
    valid = (j * _WBLK + lane) < _NIN

    def gtk(k):
        gx1 = gt_ref[:, 0 * 64 + k:0 * 64 + k + 1]
        gy1 = gt_ref[:, 1 * 64 + k:1 * 64 + k + 1]
        gx2 = gt_ref[:, 2 * 64 + k:2 * 64 + k + 1]
        gy2 = gt_ref[:, 3 * 64 + k:3 * 64 + k + 1]
        gw = gx2 - gx1 + 1.0
        gh = gy2 - gy1 + 1.0
        return gx1, gy1, gx2, gy2, gw, gh

    def overlap(k):
        gx1, gy1, gx2, gy2, gw, gh = gtk(k)
        g_area = gw * gh
        g_zero = (gw == 1.0) & (gh == 1.0)
        iw = jnp.minimum(ax2, gx2) - jnp.maximum(ax1, gx1) + 1.0
        iw = jnp.maximum(iw, 0.0)
        ih = jnp.minimum(ay2, gy2) - jnp.maximum(ay1, gy1) + 1.0
        ih = jnp.maximum(ih, 0.0)
        inter = iw * ih
        ua = a_area + g_area - inter
        ov = inter / ua
        ov = jnp.where(g_zero, 0.0, ov)
        ov = jnp.where(a_zero, -1.0, ov)
        return ov

    @pl.when(ph == 0)
    def _phase0():
        @pl.when(j == 0)
        def _init():
            gmo_ref[...] = jnp.full((_B, 128), -jnp.inf, dtype=jnp.float32)

        for k in range(_K):
            ov = overlap(k)
            ovm = jnp.where(valid, ov, -jnp.inf)
            part = jnp.max(ovm, axis=1, keepdims=True)
            gmo_ref[:, k:k + 1] = jnp.maximum(gmo_ref[:, k:k + 1], part)

    @pl.when(ph == 1)
    def _phase1():
        mx = jnp.full((_B, _WBLK), -2.0, dtype=jnp.float32)
        keep = jnp.zeros((_B, _WBLK), dtype=jnp.bool_)
        bx1 = jnp.zeros((_B, _WBLK), dtype=jnp.float32)
        by1 = jnp.zeros((_B, _WBLK), dtype=jnp.float32)
        bx2 = jnp.zeros((_B, _WBLK), dtype=jnp.float32)
        by2 = jnp.zeros((_B, _WBLK), dtype=jnp.float32)
        for k in range(_K):
            ov = overlap(k)
            gk = gmo_ref[:, k:k + 1]
            gk = jnp.where(gk == 0.0, 1e-5, gk)
            keep = keep | (ov == gk)
            upd = ov > mx
            gx1, gy1, gx2, gy2, _, _ = gtk(k)
            bx1 = jnp.where(upd, gx1, bx1)
            by1 = jnp.where(upd, gy1, by1)
            bx2 = jnp.where(upd, gx2, bx2)
            by2 = jnp.where(upd, gy2, by2)
            mx = jnp.where(upd, ov, mx)
        labels = jnp.full((_B, _WBLK), -1.0, dtype=jnp.float32)
        labels = jnp.where(mx < _NEG, 0.0, labels)
        labels = jnp.where(keep, 1.0, labels)
        labels = jnp.where(mx >= _POS, 1.0, labels)
        lab_ref[...] = labels

        bw = bx2 - bx1 + 1.0
        bh = by2 - by1 + 1.0
        bcx = bx1 + 0.5 * bw
        bcy = by1 + 0.5 * bh
        acx = ax1 + 0.5 * aw
        acy = ay1 + 0.5 * ah
        bt_ref[:, 0, :] = (bcx - acx) / aw
        bt_ref[:, 1, :] = (bcy - acy) / ah
        bt_ref[:, 2, :] = jnp.log(bw / aw)
        bt_ref[:, 3, :] = jnp.log(bh / ah)


def _tc_call(gt_in, anc_pl, interpret=False):
    return pl.pallas_call(
        _tc_body,
        grid=(2, _NB),
        in_specs=[
            pl.BlockSpec((_B, 256), lambda p, j: (0, 0)),
            pl.BlockSpec((4, _WBLK), lambda p, j: (0, j)),
        ],
        out_specs=[
            pl.BlockSpec((_B, _WBLK), lambda p, j: (0, j)),
            pl.BlockSpec((_B, 4, _WBLK), lambda p, j: (0, 0, j)),
        ],
        out_shape=[
            jax.ShapeDtypeStruct((_B, _NPAD), jnp.float32),
            jax.ShapeDtypeStruct((_B, 4, _NPAD), jnp.float32),
        ],
        scratch_shapes=[pltpu.VMEM((_B, 128), jnp.float32)],
        interpret=interpret,
    )(gt_in, anc_pl)


_SEG = 4096
_NSEG = -(-_TOTAL // _SEG)
_LABW = 34600
_NCH = _NIN // 16

_SEG_LO16 = []
_SEG_CHLEN = []
for _s in range(_NSEG):
    _a0 = _s * _SEG
    _a1 = min(_a0 + _SEG, _TOTAL)
    _lo = int(np.searchsorted(_INDS_NP, _a0, side="left")) & ~15
    _hi = int(np.searchsorted(_INDS_NP, _a1, side="left"))
    _ch = min(-(-(_hi - _lo) // 16) * 16, _NIN - _lo)
    _SEG_LO16.append(_lo)
    _SEG_CHLEN.append(_ch)
_CHMAX = max(_SEG_CHLEN)


def _count_scan(perm_v, lab_v, match_val, thresh_v):
    zero = jnp.zeros((16,), jnp.int32)

    def body(i, carry):
        cnt_v, cid_v, cbase_v = carry
        idxv = perm_v[pl.ds(i * 16, 16)]
        labv = plsc.load_gather(lab_v, [idxv])
        m = labv == match_val
        pc = plsc.all_reduce_population_count(m)
        new = (cnt_v < thresh_v) & (cnt_v + pc >= thresh_v) & (cid_v < 0)
        iv = jnp.full((16,), i, jnp.int32)
        cid_v = jnp.where(new, iv, cid_v)
        cbase_v = jnp.where(new, cnt_v, cbase_v)
        return cnt_v + pc, cid_v, cbase_v

    cnt_v, cid_v, cbase_v = lax.fori_loop(
        0, _NCH, body, (zero, zero - 1, zero))
    return cnt_v, cid_v, cbase_v


def _sc_body(lab0_hbm, psort_hbm, perm_hbm, pri_hbm, inds_hbm, btp_hbm,
             labf_hbm, btf_hbm,
             lab_v, perm_v, pri_v, inds_v, btpseg_v, labseg_v, btseg_v,
             psc_v):
    b = lax.axis_index("s")
    c = lax.axis_index("c")

    def al8(x):
        return pl.multiple_of(x, 8)

    pltpu.sync_copy(lab0_hbm.at[pl.ds(al8(b * _NPAD), _NIN)], lab_v)
    pltpu.sync_copy(perm_hbm.at[pl.ds(al8(b * _NIN), _NIN)], perm_v)
    pltpu.sync_copy(pri_hbm.at[pl.ds(al8(b * _NIN), _NIN)], pri_v)
    pltpu.sync_copy(inds_hbm, inds_v)

    def find_cut(match_val, thresh_v):
        _, cid_v, cbase_v = _count_scan(perm_v, lab_v, match_val, thresh_v)
        cid = jnp.max(cid_v)
        cbase = jnp.max(cbase_v)
        safe = jnp.maximum(cid, 0)
        idxv = perm_v[pl.ds(safe * 16, 16)]
        labv = plsc.load_gather(lab_v, [idxv])
        m = labv == match_val
        cs = plsc.cumsum(m.astype(jnp.int32)) + cbase
        pltpu.sync_copy(psort_hbm.at[pl.ds(al8(b * _NIN + safe * 16), 16)],
                        psc_v)
        sel = m & (cs == thresh_v)
        cut = jnp.max(jnp.where(sel, psc_v[...], -1.0))
        return jnp.where(cid >= 0, cut, -1.0)

    thr_fg = jnp.full((16,), _NUM_FG, jnp.int32)
    fg_cut = find_cut(1.0, thr_fg)

    def kept_body(i, acc):
        labv = lab_v[pl.ds(i * 16, 16)]
        priv = pri_v[pl.ds(i * 16, 16)]
        m = (labv == 1.0) & (priv >= fg_cut)
        return acc + plsc.all_reduce_population_count(m)

    kept_v = lax.fori_loop(0, _NCH, kept_body, jnp.zeros((16,), jnp.int32))
    num_bg = _RPN_BATCH - jnp.max(kept_v)
    thr_bg_s = jnp.maximum(num_bg, 1)
    thr_bg = jnp.full((16,), 1, jnp.int32) * thr_bg_s
    bg_cut = find_cut(0.0, thr_bg)

    for s in range(_NSEG):
        a0 = s * _SEG
        a1 = min(a0 + _SEG, _TOTAL)
        lo = _SEG_LO16[s]
        chlen = _SEG_CHLEN[s]
        lab_len = min(_SEG, _LABW - a0)
        bt_len = 4 * (a1 - a0)

        @pl.when(c == (s & 1))
        def _seg(a0=a0, a1=a1, lo=lo, chlen=chlen, lab_len=lab_len,
                 bt_len=bt_len):
            def clr_lab(i, _):
                labseg_v[pl.ds(i * 16, 16)] = jnp.full((16,), -1.0,
                                                       jnp.float32)
                return 0

            def clr_bt(i, _):
                btseg_v[pl.ds(i * 16, 16)] = jnp.zeros((16,), jnp.float32)
                return 0

            lax.fori_loop(0, _SEG // 16, clr_lab, 0)
            lax.fori_loop(0, 4 * _SEG // 16, clr_bt, 0)
            for comp in range(4):
                pltpu.sync_copy(
                    btp_hbm.at[pl.ds(al8((b * 4 + comp) * _NPAD + lo),
                                     chlen)],
                    btpseg_v.at[pl.ds(comp * _CHMAX, chlen)])

            def chunk(i, _):
                base = lo + i * 16
                indv = inds_v[pl.ds(base, 16)]
                off = indv - a0
                m = (indv >= a0) & (indv < a1)
                labv = lab_v[pl.ds(base, 16)]
                priv = pri_v[pl.ds(base, 16)]
                lf = jnp.where((labv == 1.0) & (priv < fg_cut), -1.0, labv)
                lf = jnp.where((labv == 0.0) & (priv < bg_cut), -1.0, lf)
                plsc.store_scatter(labseg_v, [off], lf, mask=m)
                for comp in range(4):
                    btv = btpseg_v[pl.ds(comp * _CHMAX + i * 16, 16)]
                    plsc.store_scatter(btseg_v, [off * 4 + comp], btv,
                                       mask=m)
                return 0

            lax.fori_loop(0, chlen // 16, chunk, 0)
            pltpu.sync_copy(labseg_v.at[pl.ds(0, lab_len)],
                            labf_hbm.at[pl.ds(al8(b * _LABW + a0), lab_len)])
            pltpu.sync_copy(btseg_v.at[pl.ds(0, bt_len)],
                            btf_hbm.at[pl.ds(al8(b * 4 * _TOTAL + 4 * a0),
                                             bt_len)])


_SC_CALL_CACHE = []


def _sc_call(*args):
    if not _SC_CALL_CACHE:
        _SC_CALL_CACHE.append(functools.partial(
            pl.kernel,
            out_type=(jax.ShapeDtypeStruct((_B * _LABW,), jnp.float32),
                      jax.ShapeDtypeStruct((_B * 4 * _TOTAL,), jnp.float32)),
            mesh=plsc.VectorSubcoreMesh(core_axis_name="c",
                                        subcore_axis_name="s"),
            scratch_types=[
                pltpu.VMEM((_NIN,), jnp.float32),
                pltpu.VMEM((_NIN,), jnp.int32),
                pltpu.VMEM((_NIN,), jnp.float32),
                pltpu.VMEM((_NIN,), jnp.int32),
                pltpu.VMEM((4 * _CHMAX,), jnp.float32),
                pltpu.VMEM((_SEG,), jnp.float32),
                pltpu.VMEM((4 * _SEG,), jnp.float32),
                pltpu.VMEM((16,), jnp.float32),
            ],
            compiler_params=pltpu.CompilerParams(needs_layout_passes=False),
        )(_sc_body))
    return _SC_CALL_CACHE[0](*args)


def kernel(gt_boxes, anchors, inds_inside):
    gt4 = gt_boxes[..., :4].astype(jnp.float32)
    gt_in = jnp.zeros((_B, 4, 64), jnp.float32)
    gt_in = gt_in.at[:, :, :_K].set(gt4.transpose(0, 2, 1)).reshape(_B, 256)
    anc_pl = jnp.zeros((4, _NPAD), jnp.float32)
    anc_pl = anc_pl.at[:, :_NIN].set(anchors.T.astype(jnp.float32))

    lab0, btp = _tc_call(gt_in, anc_pl)

    psort = jnp.asarray(_PSORT_NP.reshape(-1))
    perm = jnp.asarray(_PERM_NP.reshape(-1))
    pri = jnp.asarray(_PRI_NP.reshape(-1))
    inds32 = inds_inside.astype(jnp.int32)
    labf, btf = _sc_call(lab0.reshape(-1), psort, perm, pri, inds32,
                         btp.reshape(-1))

    labels_out = labf.reshape(_B, _LABW)[:, :_TOTAL].reshape(
        _B, _FEAT, _FEAT, _ANUM, 1)
    bt_out = btf.reshape(_B, _FEAT, _FEAT, _ANUM, 4)
    return labels_out, bt_out
